# baseline (speedup 1.0000x reference)
import functools

import jax
import jax.numpy as jnp
from jax import lax
from jax.experimental import pallas as pl
from jax.experimental.pallas import tpu as pltpu
from jax.experimental.pallas import tpu_sc as plsc

_N = 10000
_E = 320000
_D = 128
_F = 16
_NC = 2
_NS = 16
_NW = _NC * _NS
_CH = 80
_NCHUNK = 125
_EPW = _NCHUNK * _CH
_RPS = 632
_NP = _NS * _RPS


_U = 5
_K = 4


def _spmm_sc(table, src, dst, zeros):
    mesh = plsc.VectorSubcoreMesh(core_axis_name="c", subcore_axis_name="s")

    @functools.partial(
        pl.kernel,
        mesh=mesh,
        out_type=jax.ShapeDtypeStruct((_NC, _NP, _F), jnp.float32),
        scratch_types=[
            pltpu.VMEM((_NCHUNK, _CH), jnp.int32),
            pltpu.VMEM((_NCHUNK, _CH), jnp.int32),
            pltpu.VMEM((_U, _CH, _F), jnp.float32),
            pltpu.VMEM_SHARED((_NP, _F), jnp.float32),
            [pltpu.SemaphoreType.DMA] * _U,
            [pltpu.SemaphoreType.DMA] * _U,
            [pltpu.SemaphoreType.DMA] * 3,
        ],
        compiler_params=pltpu.CompilerParams(use_tc_tiling_on_sc=False),
    )
    def k(table_h, srcs_h, dsts_h, zeros_h, out_h, sidx2, didx2, rows,
          acc, gsem, ssem, psem):
        c = lax.axis_index("c")
        s = lax.axis_index("s")
        wid = s * _NC + c
        pltpu.async_copy(zeros_h,
                         acc.at[pl.ds(s * _RPS, _RPS)], psem[0])
        pltpu.async_copy(srcs_h.at[wid], sidx2, psem[1])
        pltpu.async_copy(dsts_h.at[wid], didx2, psem[2])
        pltpu.make_async_copy(srcs_h.at[wid], sidx2, psem[1]).wait()
        pltpu.make_async_copy(dsts_h.at[wid], didx2, psem[2]).wait()

        def gather_start(chunk, b):
            pltpu.async_copy(table_h.at[sidx2.at[chunk]], rows.at[b],
                             gsem[b])

        def gather_wait(chunk, b):
            pltpu.make_async_copy(table_h.at[sidx2.at[chunk]], rows.at[b],
                                  gsem[b]).wait()

        def scatter_start(chunk, b):
            pltpu.async_copy(rows.at[b], acc.at[didx2.at[chunk]], ssem[b],
                             add=True)

        def scatter_wait(chunk, b):
            pltpu.make_async_copy(rows.at[b], acc.at[didx2.at[chunk]],
                                  ssem[b]).wait()

        for b in range(_K):
            gather_start(b, b)
        pltpu.make_async_copy(zeros_h,
                              acc.at[pl.ds(s * _RPS, _RPS)], psem[0]).wait()
        plsc.subcore_barrier()

        def outer(g, carry):
            for b in range(_U):
                cg = g * _U + b + _K
                cs = g * _U + b
                bs = b
                bg = (b + _K) % _U

                @pl.when(cg < _NCHUNK)
                def _():
                    @pl.when(cg >= _U)
                    def _():
                        scatter_wait(cg - _U, bg)
                    gather_start(cg, bg)

                gather_wait(cs, bs)
                scatter_start(cs, bs)
            return carry

        lax.fori_loop(0, _NCHUNK // _U, outer, 0)
        for b in range(_U):
            scatter_wait(_NCHUNK - _U + b, (_NCHUNK - _U + b) % _U)
        plsc.subcore_barrier()
        pltpu.sync_copy(acc.at[pl.ds(s * _RPS, _RPS)],
                        out_h.at[c, pl.ds(s * _RPS, _RPS)])

    return k(table, src, dst, zeros)


def _pad_edges(src, dst):
    pad = _EPW - _E // _NW
    srcs3 = jnp.pad(src.reshape(_NW, _E // _NW),
                    ((0, 0), (0, pad))).reshape(_NW, _NCHUNK, _CH)
    dsts3 = jnp.pad(dst.reshape(_NW, _E // _NW), ((0, 0), (0, pad)),
                    constant_values=_N).reshape(_NW, _NCHUNK, _CH)
    return srcs3, dsts3


def _tc_xw(X, W1):
    br = _N

    def body(x_ref, w_ref, o_ref):
        o_ref[...] = jnp.dot(x_ref[...], w_ref[...],
                             preferred_element_type=jnp.float32)

    return pl.pallas_call(
        body,
        grid=(_N // br,),
        in_specs=[
            pl.BlockSpec((br, _D), lambda i: (i, 0)),
            pl.BlockSpec((_D, _F), lambda i: (0, 0)),
        ],
        out_specs=pl.BlockSpec((br, _F), lambda i: (i, 0)),
        out_shape=jax.ShapeDtypeStruct((_N, _F), jnp.float32),
    )(X, W1)


def _tc_sum_relu(p):
    br = _N

    def body(p_ref, o_ref):
        o_ref[...] = jnp.maximum(p_ref[0] + p_ref[1], 0.0)

    return pl.pallas_call(
        body,
        grid=(_N // br,),
        in_specs=[pl.BlockSpec((_NC, br, _F), lambda i: (0, i, 0))],
        out_specs=pl.BlockSpec((br, _F), lambda i: (i, 0)),
        out_shape=jax.ShapeDtypeStruct((_N, _F), jnp.float32),
    )(p)


def _tc_sum_mm2(q, W2, W3):
    br = _N
    h2 = W2.shape[1]

    def body(q_ref, w2_ref, w3_ref, o1_ref, o2_ref):
        s = q_ref[0] + q_ref[1]
        o1_ref[...] = jnp.dot(s, w2_ref[...],
                              preferred_element_type=jnp.float32)
        o2_ref[...] = jnp.dot(s, w3_ref[...],
                              preferred_element_type=jnp.float32)

    return pl.pallas_call(
        body,
        grid=(_N // br,),
        in_specs=[
            pl.BlockSpec((_NC, br, _F), lambda i: (0, i, 0)),
            pl.BlockSpec((_F, h2), lambda i: (0, 0)),
            pl.BlockSpec((_F, h2), lambda i: (0, 0)),
        ],
        out_specs=[
            pl.BlockSpec((br, h2), lambda i: (i, 0)),
            pl.BlockSpec((br, h2), lambda i: (i, 0)),
        ],
        out_shape=[
            jax.ShapeDtypeStruct((_N, h2), jnp.float32),
            jax.ShapeDtypeStruct((_N, h2), jnp.float32),
        ],
    )(q, W2, W3)


def kernel(X, edge_index, W1, W2, W3):
    srcs3, dsts3 = _pad_edges(edge_index[0], edge_index[1])
    zeros = jnp.zeros((_RPS, _F), jnp.float32)
    t1 = _tc_xw(X, W1)
    p = _spmm_sc(t1, srcs3, dsts3, zeros)
    latent = _tc_sum_relu(p)
    q = _spmm_sc(latent, srcs3, dsts3, zeros)
    return _tc_sum_mm2(q, W2, W3)


# speedup vs baseline: 1.0189x; 1.0189x over previous
import functools

import jax
import jax.numpy as jnp
from jax import lax
from jax.experimental import pallas as pl
from jax.experimental.pallas import tpu as pltpu
from jax.experimental.pallas import tpu_sc as plsc

_N = 10000
_E = 320000
_D = 128
_F = 16
_NC = 2
_NS = 16
_NW = _NC * _NS
_CH = 80
_NCHUNK = 125
_EPW = _NCHUNK * _CH
_RPS = 632
_NP = _NS * _RPS


_U = 5
_K = 4


def _spmm_sc(table, src, dst, zeros):
    mesh = plsc.VectorSubcoreMesh(core_axis_name="c", subcore_axis_name="s")

    @functools.partial(
        pl.kernel,
        mesh=mesh,
        out_type=jax.ShapeDtypeStruct((_NC, _NP, _F), jnp.float32),
        scratch_types=[
            pltpu.VMEM((_NCHUNK, _CH), jnp.int32),
            pltpu.VMEM((_NCHUNK, _CH), jnp.int32),
            pltpu.VMEM((_U, _CH, _F), jnp.float32),
            pltpu.VMEM_SHARED((_NP, _F), jnp.float32),
            [pltpu.SemaphoreType.DMA] * _U,
            [pltpu.SemaphoreType.DMA] * _U,
            [pltpu.SemaphoreType.DMA] * 3,
        ],
        compiler_params=pltpu.CompilerParams(use_tc_tiling_on_sc=False),
    )
    def k(table_h, srcs_h, dsts_h, zeros_h, out_h, sidx2, didx2, rows,
          acc, gsem, ssem, psem):
        c = lax.axis_index("c")
        s = lax.axis_index("s")
        wid = s * _NC + c
        pltpu.async_copy(zeros_h.at[pl.ds(s * _RPS, _RPS)],
                         acc.at[pl.ds(s * _RPS, _RPS)], psem[0])
        pltpu.async_copy(srcs_h.at[wid], sidx2, psem[1])
        pltpu.async_copy(dsts_h.at[wid], didx2, psem[2])
        pltpu.make_async_copy(srcs_h.at[wid], sidx2, psem[1]).wait()
        pltpu.make_async_copy(dsts_h.at[wid], didx2, psem[2]).wait()

        def gather_start(chunk, b):
            pltpu.async_copy(table_h.at[sidx2.at[chunk]], rows.at[b],
                             gsem[b])

        def gather_wait(chunk, b):
            pltpu.make_async_copy(table_h.at[sidx2.at[chunk]], rows.at[b],
                                  gsem[b]).wait()

        def scatter_start(chunk, b):
            pltpu.async_copy(rows.at[b], acc.at[didx2.at[chunk]], ssem[b],
                             add=True)

        def scatter_wait(chunk, b):
            pltpu.make_async_copy(rows.at[b], acc.at[didx2.at[chunk]],
                                  ssem[b]).wait()

        for b in range(_K):
            gather_start(b, b)
        pltpu.make_async_copy(zeros_h.at[pl.ds(s * _RPS, _RPS)],
                              acc.at[pl.ds(s * _RPS, _RPS)], psem[0]).wait()
        plsc.subcore_barrier()

        def outer(g, carry):
            for b in range(_U):
                cg = g * _U + b + _K
                cs = g * _U + b
                bs = b
                bg = (b + _K) % _U

                @pl.when(cg < _NCHUNK)
                def _():
                    @pl.when(cg >= _U)
                    def _():
                        scatter_wait(cg - _U, bg)
                    gather_start(cg, bg)

                gather_wait(cs, bs)
                scatter_start(cs, bs)
            return carry

        lax.fori_loop(0, _NCHUNK // _U, outer, 0)
        for b in range(_U):
            scatter_wait(_NCHUNK - _U + b, (_NCHUNK - _U + b) % _U)
        plsc.subcore_barrier()
        pltpu.sync_copy(acc.at[pl.ds(s * _RPS, _RPS)],
                        out_h.at[c, pl.ds(s * _RPS, _RPS)])

    return k(table, src, dst, zeros)


def _pad_edges(src, dst):
    pad = _EPW - _E // _NW
    srcs3 = jnp.pad(src.reshape(_NW, _E // _NW),
                    ((0, 0), (0, pad))).reshape(_NW, _NCHUNK, _CH)
    dsts3 = jnp.pad(dst.reshape(_NW, _E // _NW), ((0, 0), (0, pad)),
                    constant_values=_N).reshape(_NW, _NCHUNK, _CH)
    return srcs3, dsts3


def _tc_xw(X, W1):
    br = _N

    def body(x_ref, w_ref, o_ref):
        o_ref[...] = jnp.dot(x_ref[...], w_ref[...],
                             preferred_element_type=jnp.float32)

    return pl.pallas_call(
        body,
        grid=(_N // br,),
        in_specs=[
            pl.BlockSpec((br, _D), lambda i: (i, 0)),
            pl.BlockSpec((_D, _F), lambda i: (0, 0)),
        ],
        out_specs=pl.BlockSpec((br, _F), lambda i: (i, 0)),
        out_shape=jax.ShapeDtypeStruct((_N, _F), jnp.float32),
    )(X, W1)


def _tc_sum_relu(p):
    br = _N

    def body(p_ref, o_ref):
        o_ref[...] = jnp.maximum(p_ref[0] + p_ref[1], 0.0)

    return pl.pallas_call(
        body,
        grid=(_N // br,),
        in_specs=[pl.BlockSpec((_NC, br, _F), lambda i: (0, i, 0))],
        out_specs=pl.BlockSpec((br, _F), lambda i: (i, 0)),
        out_shape=jax.ShapeDtypeStruct((_N, _F), jnp.float32),
    )(p)


def _tc_sum_mm2(q, W2, W3):
    br = _N
    h2 = W2.shape[1]

    def body(q_ref, w2_ref, w3_ref, o1_ref, o2_ref):
        s = q_ref[0] + q_ref[1]
        o1_ref[...] = jnp.dot(s, w2_ref[...],
                              preferred_element_type=jnp.float32)
        o2_ref[...] = jnp.dot(s, w3_ref[...],
                              preferred_element_type=jnp.float32)

    return pl.pallas_call(
        body,
        grid=(_N // br,),
        in_specs=[
            pl.BlockSpec((_NC, br, _F), lambda i: (0, i, 0)),
            pl.BlockSpec((_F, h2), lambda i: (0, 0)),
            pl.BlockSpec((_F, h2), lambda i: (0, 0)),
        ],
        out_specs=[
            pl.BlockSpec((br, h2), lambda i: (i, 0)),
            pl.BlockSpec((br, h2), lambda i: (i, 0)),
        ],
        out_shape=[
            jax.ShapeDtypeStruct((_N, h2), jnp.float32),
            jax.ShapeDtypeStruct((_N, h2), jnp.float32),
        ],
    )(q, W2, W3)


def kernel(X, edge_index, W1, W2, W3):
    srcs3, dsts3 = _pad_edges(edge_index[0], edge_index[1])
    zeros = jnp.zeros((_NP, _F), jnp.float32)
    t1 = _tc_xw(X, W1)
    p = _spmm_sc(t1, srcs3, dsts3, zeros)
    latent = _tc_sum_relu(p)
    q = _spmm_sc(latent, srcs3, dsts3, zeros)
    return _tc_sum_mm2(q, W2, W3)
